# baseline (speedup 1.0000x reference)
import functools

import jax
import jax.numpy as jnp
from jax import lax
from jax.experimental import pallas as pl
from jax.experimental.pallas import tpu as pltpu
from jax.experimental.pallas import tpu_sc as plsc

E = 3200000
NW = 32
C = 640
SUB = 128
NSUB = C // SUB
NCHUNK = E // C
FP = 16
SEG = 8
SEG_ROWS = E // SEG
CSEG = SEG_ROWS // C
F_XS, F_XT, F_E, F_U = 10, 5, 10, 10
N_GRAPHS = 64
BLK = 10000
GRID_R = SEG_ROWS // BLK


def _sc_gather(src2d, tgt2d, b2d, xs_t, xt_t, u_t):
    mesh = plsc.VectorSubcoreMesh(core_axis_name="c", subcore_axis_name="s")

    @functools.partial(
        pl.kernel,
        mesh=mesh,
        out_type=jax.ShapeDtypeStruct((SEG_ROWS, SEG * FP), jnp.float32),
        scratch_types=[
            pltpu.VMEM((C,), jnp.int32),
            pltpu.VMEM((C,), jnp.int32),
            pltpu.VMEM((C,), jnp.int32),
            pltpu.VMEM((C, FP), jnp.float32),
            pltpu.VMEM((C, FP), jnp.float32),
            pltpu.VMEM((C, FP), jnp.float32),
            pltpu.SemaphoreType.DMA,
        ],
        compiler_params=pltpu.CompilerParams(use_tc_tiling_on_sc=False),
    )
    def body(src_hbm, tgt_hbm, b_hbm, xs_hbm, xt_hbm, u_hbm, g_hbm,
             src_v, tgt_v, b_v, gs_v, gt_v, gu_v, sem):
        w = lax.axis_index("s") * 2 + lax.axis_index("c")
        n_w = (NCHUNK - w + NW - 1) // NW

        def chunk(i, carry):
            k = w + i * NW
            pltpu.sync_copy(src_hbm.at[k], src_v)
            pltpu.sync_copy(tgt_hbm.at[k], tgt_v)
            pltpu.sync_copy(b_hbm.at[k], b_v)
            copies = []
            for j in range(NSUB):
                s = j * SUB
                copies.append(pltpu.async_copy(
                    xs_hbm.at[src_v.at[pl.ds(s, SUB)]],
                    gs_v.at[pl.ds(s, SUB)], sem))
                copies.append(pltpu.async_copy(
                    xt_hbm.at[tgt_v.at[pl.ds(s, SUB)]],
                    gt_v.at[pl.ds(s, SUB)], sem))
                copies.append(pltpu.async_copy(
                    u_hbm.at[b_v.at[pl.ds(s, SUB)]],
                    gu_v.at[pl.ds(s, SUB)], sem))
            for cp in copies:
                cp.wait()

            def merge(r, carry2):
                gs_v[r] = gs_v[r] + gt_v[r] + gu_v[r]
                return carry2

            lax.fori_loop(0, C, merge, 0)
            seg = k // CSEG
            row0 = (k % CSEG) * C
            pltpu.sync_copy(gs_v,
                            g_hbm.at[pl.ds(row0, C), pl.ds(seg * FP, FP)])
            return carry

        lax.fori_loop(0, n_w, chunk, 0)

    return body(src2d, tgt2d, b2d, xs_t, xt_t, u_t)


def _tc_body(g_ref, ea_ref, w1_ref, w2_ref, b2_ref, o_ref):
    s = pl.program_id(1)
    rows = lax.broadcasted_iota(jnp.int32, (SEG * FP, FP), 0)
    cols = lax.broadcasted_iota(jnp.int32, (SEG * FP, FP), 1)
    sel = (rows == s * FP + cols).astype(jnp.float32)
    z = jnp.dot(g_ref[...], sel, preferred_element_type=jnp.float32)
    z = z + jnp.dot(ea_ref[...], w1_ref[...], preferred_element_type=jnp.float32)
    h1 = jnp.where(z >= 0, z, 0.1 * z)
    o_ref[...] = (jnp.dot(h1, w2_ref[...], preferred_element_type=jnp.float32)
                  + b2_ref[...])


def _tc_mlp(g2, ea, w1et, w2tp, b2r):
    return pl.pallas_call(
        _tc_body,
        grid=(GRID_R, SEG),
        in_specs=[
            pl.BlockSpec((BLK, SEG * FP), lambda r, s: (r, 0)),
            pl.BlockSpec((BLK, F_E), lambda r, s: (s * GRID_R + r, 0)),
            pl.BlockSpec((F_E, FP), lambda r, s: (0, 0)),
            pl.BlockSpec((FP, F_E), lambda r, s: (0, 0)),
            pl.BlockSpec((1, F_E), lambda r, s: (0, 0)),
        ],
        out_specs=pl.BlockSpec((BLK, F_E), lambda r, s: (s * GRID_R + r, 0)),
        out_shape=jax.ShapeDtypeStruct((E, F_E), jnp.float32),
    )(g2, ea, w1et, w2tp, b2r)


def kernel(x_s, x_t, edge_index, edge_attr, u, batch_e, W1, b1, W2, b2):
    src2d = edge_index[0].reshape(NCHUNK, C)
    tgt2d = edge_index[1].reshape(NCHUNK, C)
    b2d = batch_e.reshape(NCHUNK, C)

    xs1 = x_s @ W1[:, :F_XS].T
    xt1 = x_t @ W1[:, F_XS:F_XS + F_XT].T
    u1 = u @ W1[:, F_XS + F_XT + F_E:].T + b1

    def padw(a):
        return jnp.pad(a, ((0, 0), (0, FP - a.shape[1])))

    g2 = _sc_gather(src2d, tgt2d, b2d, padw(xs1), padw(xt1), padw(u1))

    w1et = padw(W1[:, F_XS + F_XT:F_XS + F_XT + F_E].T)
    w2tp = jnp.pad(W2.T, ((0, FP - F_E), (0, 0)))
    return _tc_mlp(g2, edge_attr, w1et, w2tp, b2.reshape(1, F_E))
